# baseline (speedup 1.0000x reference)
import functools

import jax
import jax.numpy as jnp
from jax import lax
from jax.experimental import pallas as pl
from jax.experimental.pallas import tpu as pltpu
from jax.experimental.pallas import tpu_sc as plsc

_NC = 2
_NS = 16
_NW = _NC * _NS
_C = 16


def _sc_body(x_hbm, pe_hbm, out_hbm, pe_v, x_v, *, B, S, D):
    wid = lax.axis_index("s") * _NC + lax.axis_index("c")
    ppw = S // _NW
    n_chunks = ppw // _C
    base = wid * ppw * D

    def chunk_body(ci, _):
        off = base + ci * (_C * D)
        pltpu.sync_copy(pe_hbm.at[pl.ds(off, _C * D)], pe_v)
        for b in range(B):
            xoff = b * (S * D) + off
            pltpu.sync_copy(x_hbm.at[pl.ds(xoff, _C * D)], x_v)

            @plsc.parallel_loop(0, _C * D, step=16, unroll=8)
            def _(i):
                plsc.addupdate(x_v.at[pl.ds(i, 16)], pe_v[pl.ds(i, 16)])

            pltpu.sync_copy(x_v, out_hbm.at[pl.ds(xoff, _C * D)])
        return ()

    lax.fori_loop(0, n_chunks, chunk_body, ())


def kernel(x, pe):
    B, S, D = x.shape
    xf = x.reshape(B * S * D)
    pef = pe[:S].reshape(S * D)

    mesh = plsc.VectorSubcoreMesh(core_axis_name="c", subcore_axis_name="s")
    k = pl.kernel(
        functools.partial(_sc_body, B=B, S=S, D=D),
        out_type=jax.ShapeDtypeStruct((B * S * D,), jnp.float32),
        mesh=mesh,
        scratch_types=[
            pltpu.VMEM((_C * D,), jnp.float32),
            pltpu.VMEM((_C * D,), jnp.float32),
        ],
    )
    return k(xf, pef).reshape(B, S, D)


# speedup vs baseline: 1.3032x; 1.3032x over previous
import functools

import jax
import jax.numpy as jnp
from jax import lax
from jax.experimental import pallas as pl
from jax.experimental.pallas import tpu as pltpu
from jax.experimental.pallas import tpu_sc as plsc

_NC = 2
_NS = 16
_NW = _NC * _NS
_C = 8
_RING = 3


def _sc_body(x_hbm, pe_hbm, out_hbm, pe_v, x_v, sem_pe, sem_ld, sem_st,
             *, B, S, D):
    wid = lax.axis_index("s") * _NC + lax.axis_index("c")
    ppw = S // _NW
    nch = ppw // _C
    CW = _C * D
    base = wid * ppw * D

    pe_h = [None, None]
    ld_h = [[None] * B for _ in range(_RING)]
    st_h = [[None] * B for _ in range(_RING)]

    def start_pe(g):
        p = g % 2
        pe_h[p] = pltpu.async_copy(
            pe_hbm.at[pl.ds(base + g * CW, CW)], pe_v.at[p], sem_pe.at[p])

    def start_ld(g, b):
        r = g % _RING
        ld_h[r][b] = pltpu.async_copy(
            x_hbm.at[pl.ds(b * (S * D) + base + g * CW, CW)],
            x_v.at[r * B + b], sem_ld.at[r * B + b])

    def start_st(g, b):
        r = g % _RING
        st_h[r][b] = pltpu.async_copy(
            x_v.at[r * B + b],
            out_hbm.at[pl.ds(b * (S * D) + base + g * CW, CW)],
            sem_st.at[r * B + b])

    start_pe(0)
    for b in range(B):
        start_ld(0, b)
    if nch > 1:
        start_pe(1)
        for b in range(B):
            start_ld(1, b)

    for g in range(nch):
        p, r = g % 2, g % _RING
        pe_h[p].wait()
        for b in range(B):
            ld_h[r][b].wait()

        @plsc.parallel_loop(0, CW, step=16, unroll=4)
        def _(i):
            pv = pe_v.at[p][pl.ds(i, 16)]
            for b in range(B):
                plsc.addupdate(x_v.at[r * B + b].at[pl.ds(i, 16)], pv)

        for b in range(B):
            start_st(g, b)

        if g + 2 < nch:
            r2 = (g + 2) % _RING
            for b in range(B):
                if g >= 1:
                    st_h[r2][b].wait()
                start_ld(g + 2, b)
            start_pe(g + 2)

    for g in range(max(nch - 2, 0), nch):
        for b in range(B):
            st_h[g % _RING][b].wait()


def kernel(x, pe):
    B, S, D = x.shape
    xf = x.reshape(B * S * D)
    pef = pe[:S].reshape(S * D)

    mesh = plsc.VectorSubcoreMesh(core_axis_name="c", subcore_axis_name="s")
    k = pl.kernel(
        functools.partial(_sc_body, B=B, S=S, D=D),
        out_type=jax.ShapeDtypeStruct((B * S * D,), jnp.float32),
        mesh=mesh,
        compiler_params=pltpu.CompilerParams(use_tc_tiling_on_sc=False),
        scratch_types=[
            pltpu.VMEM((2, _C * D), jnp.float32),
            pltpu.VMEM((_RING * B, _C * D), jnp.float32),
            pltpu.SemaphoreType.DMA((2,)),
            pltpu.SemaphoreType.DMA((_RING * B,)),
            pltpu.SemaphoreType.DMA((_RING * B,)),
        ],
    )
    return k(xf, pef).reshape(B, S, D)


# speedup vs baseline: 3.7251x; 2.8584x over previous
import functools

import jax
import jax.numpy as jnp
from jax import lax
from jax.experimental import pallas as pl
from jax.experimental.pallas import tpu as pltpu
from jax.experimental.pallas import tpu_sc as plsc

_NC = 2
_NS = 16
_NW = _NC * _NS
_C = 8
_RING = 3


def _sc_body(x_hbm, pe_hbm, out_hbm, pe_v, x_v, sem_pe, sem_ld, sem_st,
             *, B, S, D):
    wid = lax.axis_index("s") * _NC + lax.axis_index("c")
    ppw = S // _NW
    nch = ppw // _C
    base = wid * ppw

    pe_h = [None, None]
    ld_h = [[None] * B for _ in range(_RING)]
    st_h = [[None] * B for _ in range(_RING)]

    def start_pe(g):
        p = g % 2
        pe_h[p] = pltpu.async_copy(
            pe_hbm.at[pl.ds(base + g * _C, _C)], pe_v.at[p], sem_pe.at[p])

    def start_ld(g, b):
        r = g % _RING
        ld_h[r][b] = pltpu.async_copy(
            x_hbm.at[b, pl.ds(base + g * _C, _C)],
            x_v.at[r * B + b], sem_ld.at[r * B + b])

    def start_st(g, b):
        r = g % _RING
        st_h[r][b] = pltpu.async_copy(
            x_v.at[r * B + b],
            out_hbm.at[b, pl.ds(base + g * _C, _C)],
            sem_st.at[r * B + b])

    start_pe(0)
    for b in range(B):
        start_ld(0, b)
    if nch > 1:
        start_pe(1)
        for b in range(B):
            start_ld(1, b)

    for g in range(nch):
        p, r = g % 2, g % _RING
        pe_h[p].wait()
        for b in range(B):
            ld_h[r][b].wait()

        @plsc.parallel_loop(0, D, step=16, unroll=2)
        def _(i):
            for rw in range(_C):
                pv = pe_v.at[p][rw, pl.ds(i, 16)]
                for b in range(B):
                    plsc.addupdate(x_v.at[r * B + b].at[rw, pl.ds(i, 16)], pv)

        for b in range(B):
            start_st(g, b)

        if g + 2 < nch:
            for b in range(B):
                if g >= 1:
                    st_h[(g + 2) % _RING][b].wait()
                start_ld(g + 2, b)
            start_pe(g + 2)

    for g in range(max(nch - 2, 0), nch):
        for b in range(B):
            st_h[g % _RING][b].wait()


def kernel(x, pe):
    B, S, D = x.shape

    mesh = plsc.VectorSubcoreMesh(core_axis_name="c", subcore_axis_name="s")
    k = pl.kernel(
        functools.partial(_sc_body, B=B, S=S, D=D),
        out_type=jax.ShapeDtypeStruct((B, S, D), jnp.float32),
        mesh=mesh,
        scratch_types=[
            pltpu.VMEM((2, _C, D), jnp.float32),
            pltpu.VMEM((_RING * B, _C, D), jnp.float32),
            pltpu.SemaphoreType.DMA((2,)),
            pltpu.SemaphoreType.DMA((_RING * B,)),
            pltpu.SemaphoreType.DMA((_RING * B,)),
        ],
    )
    return k(x, pe[:S])


# speedup vs baseline: 3.8143x; 1.0240x over previous
import functools

import jax
import jax.numpy as jnp
from jax import lax
from jax.experimental import pallas as pl
from jax.experimental.pallas import tpu as pltpu
from jax.experimental.pallas import tpu_sc as plsc

_NC = 2
_NS = 16
_NW = _NC * _NS
_C = 8
_RING = 3


def _sc_body(x_hbm, pe_hbm, out_hbm, pe_v, x_v, sem_pe, sem_ld, sem_st,
             *, B, S, D):
    wid = lax.axis_index("s") * _NC + lax.axis_index("c")
    ppw = S // _NW
    nch = ppw // _C
    base = wid * ppw

    pe_h = [None, None]
    ld_h = [None] * _RING
    st_h = [None] * _RING

    def start_pe(g):
        p = g % 2
        pe_h[p] = pltpu.async_copy(
            pe_hbm.at[pl.ds(base + g * _C, _C)], pe_v.at[p], sem_pe.at[p])

    def start_ld(g):
        r = g % _RING
        ld_h[r] = pltpu.async_copy(
            x_hbm.at[:, pl.ds(base + g * _C, _C)], x_v.at[r], sem_ld.at[r])

    def start_st(g):
        r = g % _RING
        st_h[r] = pltpu.async_copy(
            x_v.at[r], out_hbm.at[:, pl.ds(base + g * _C, _C)], sem_st.at[r])

    start_pe(0)
    start_ld(0)
    if nch > 1:
        start_pe(1)
        start_ld(1)

    for g in range(nch):
        p, r = g % 2, g % _RING
        pe_h[p].wait()
        ld_h[r].wait()

        @plsc.parallel_loop(0, D, step=16, unroll=1)
        def _(i):
            for rw in range(_C):
                pv = pe_v.at[p][rw, pl.ds(i, 16)]
                for b in range(B):
                    plsc.addupdate(x_v.at[r, b, rw, pl.ds(i, 16)], pv)

        start_st(g)

        if g + 2 < nch:
            if g >= 1:
                st_h[(g + 2) % _RING].wait()
            start_ld(g + 2)
            start_pe(g + 2)

    for g in range(max(nch - 2, 0), nch):
        st_h[g % _RING].wait()


def kernel(x, pe):
    B, S, D = x.shape

    mesh = plsc.VectorSubcoreMesh(core_axis_name="c", subcore_axis_name="s")
    k = pl.kernel(
        functools.partial(_sc_body, B=B, S=S, D=D),
        out_type=jax.ShapeDtypeStruct((B, S, D), jnp.float32),
        mesh=mesh,
        scratch_types=[
            pltpu.VMEM((2, _C, D), jnp.float32),
            pltpu.VMEM((_RING, B, _C, D), jnp.float32),
            pltpu.SemaphoreType.DMA((2,)),
            pltpu.SemaphoreType.DMA((_RING,)),
            pltpu.SemaphoreType.DMA((_RING,)),
        ],
    )
    return k(x, pe[:S])
